# baseline (speedup 1.0000x reference)
import functools

import jax
import jax.numpy as jnp
from jax import lax
from jax.experimental import pallas as pl
from jax.experimental.pallas import tpu as pltpu
from jax.experimental.pallas import tpu_sc as plsc

B, S, T, D, L = 64, 50, 200, 64, 5000
ROWS_PER_BATCH = S * T
N_ROWS = B * ROWS_PER_BATCH
NUM_WORKERS = 32
SC_BATCHES = 32
TC_BATCHES = B - SC_BATCHES
BATCHES_PER_WORKER = SC_BATCHES // NUM_WORKERS
CHUNK = 400
GCHUNK = 80
NGATHER = CHUNK // GCHUNK
NCHUNKS = ROWS_PER_BATCH // CHUNK
LANES = 16
TC_BLK = 2000


def _sc_body(in_hbm, ts_hbm, table_hbm, out_hbm,
             ts_buf, idx_bufs, in_bufs, table_spm,
             sem_in, sem_out, sem_g):
    wid = lax.axis_index("s") * 2 + lax.axis_index("c")

    @pl.when(lax.axis_index("s") == 0)
    def _():
        pltpu.sync_copy(table_hbm, table_spm)

    plsc.subcore_barrier()

    for bb in range(BATCHES_PER_WORKER):
        b = wid * BATCHES_PER_WORKER + bb
        row0 = b * ROWS_PER_BATCH

        pltpu.sync_copy(ts_hbm.at[pl.ds(row0, ROWS_PER_BATCH)], ts_buf)

        def min_step(j, m):
            return jnp.minimum(m, ts_buf[pl.ds(j * LANES, LANES)])

        m0 = jnp.full((LANES,), jnp.iinfo(jnp.int32).max, dtype=jnp.int32)
        m = lax.fori_loop(0, ROWS_PER_BATCH // LANES, min_step, m0)
        iota = lax.broadcasted_iota(jnp.int32, (LANES,), 0)
        for k in (8, 4, 2, 1):
            perm = jnp.take_along_axis(m, iota ^ k, axis=0,
                                       mode="promise_in_bounds")
            m = jnp.minimum(m, perm)
        min_splat = m

        def compute_idx(j, p):
            for u in range(CHUNK // LANES):
                idx_bufs[p][pl.ds(u * LANES, LANES)] = (
                    ts_buf[pl.ds(j * CHUNK + u * LANES, LANES)] - min_splat)

        def in_copy(j, p):
            return pltpu.make_async_copy(
                in_hbm.at[pl.ds(row0 + j * CHUNK, CHUNK)],
                in_bufs[p], sem_in[p])

        def out_copy(j, p):
            return pltpu.make_async_copy(
                in_bufs[p], out_hbm.at[pl.ds(row0 + j * CHUNK, CHUNK)],
                sem_out[p])

        def chunk_body(j, p, first):
            in_copy(j, p).wait()
            descs = [
                pltpu.make_async_copy(
                    table_spm.at[idx_bufs[p].at[pl.ds(g * GCHUNK, GCHUNK)]],
                    in_bufs[p].at[pl.ds(g * GCHUNK, GCHUNK)],
                    sem_g)
                for g in range(NGATHER)
            ]
            for d in descs:
                d.start(add=True)
            q = 1 - p

            @pl.when(j + 1 < NCHUNKS)
            def _():
                if not first:
                    out_copy(j, q).wait()
                compute_idx(j + 1, q)
                in_copy(j + 1, q).start()

            for d in descs:
                d.wait()

            out_copy(j, p).start()

        compute_idx(0, 0)
        in_copy(0, 0).start()
        chunk_body(0, 0, first=True)

        def pair(i, carry):
            chunk_body(2 * i - 1, 1, first=False)
            chunk_body(2 * i, 0, first=False)
            return carry

        lax.fori_loop(1, (NCHUNKS + 1) // 2, pair, 0)

        out_copy(NCHUNKS - 2, 1).wait()
        out_copy(NCHUNKS - 1, 0).wait()


def _run_sc(in2d, ts1d, pos_encoding):
    mesh = plsc.VectorSubcoreMesh(core_axis_name="c", subcore_axis_name="s")
    run = pl.kernel(
        _sc_body,
        out_type=jax.ShapeDtypeStruct((SC_BATCHES * ROWS_PER_BATCH, D),
                                      jnp.float32),
        mesh=mesh,
        scratch_types=[
            pltpu.VMEM((ROWS_PER_BATCH,), jnp.int32),
            [pltpu.VMEM((CHUNK,), jnp.int32) for _ in range(2)],
            [pltpu.VMEM((CHUNK, D), jnp.float32) for _ in range(2)],
            pltpu.VMEM_SHARED((L, D), jnp.float32),
            [pltpu.SemaphoreType.DMA for _ in range(2)],
            [pltpu.SemaphoreType.DMA for _ in range(2)],
            pltpu.SemaphoreType.DMA,
        ],
        compiler_params=pltpu.CompilerParams(use_tc_tiling_on_sc=False),
    )
    return run(in2d, ts1d, pos_encoding)


def _tc_body(ts_ref, ts_blk_ref, in_ref, out_ref):
    min_b = jnp.min(ts_ref[0, 0])
    delta = (ts_blk_ref[0, 0] - min_b).astype(jnp.float32)
    d_iota = lax.broadcasted_iota(jnp.int32, (TC_BLK, D), 1)
    is_odd = (d_iota % 2).astype(jnp.float32)
    pair = (d_iota - d_iota % 2).astype(jnp.float32)
    factor = jnp.exp(pair * (-jnp.log(10000.0) / D))
    arg = delta[:, None] * factor + is_odd * (jnp.pi / 2)
    out_ref[...] = in_ref[...] + jnp.sin(arg)[None]


def _run_tc(in3, ts2, ts_blocks):
    grid = (TC_BATCHES, ROWS_PER_BATCH // TC_BLK)
    return pl.pallas_call(
        _tc_body,
        grid=grid,
        in_specs=[
            pl.BlockSpec((1, 1, ROWS_PER_BATCH),
                         lambda b, j: (SC_BATCHES + b, 0, 0)),
            pl.BlockSpec((1, 1, TC_BLK),
                         lambda b, j: ((SC_BATCHES + b) * (ROWS_PER_BATCH
                                                           // TC_BLK) + j,
                                       0, 0)),
            pl.BlockSpec((1, TC_BLK, D),
                         lambda b, j: (SC_BATCHES + b, j, 0)),
        ],
        out_specs=pl.BlockSpec((1, TC_BLK, D), lambda b, j: (b, j, 0)),
        out_shape=jax.ShapeDtypeStruct((TC_BATCHES, ROWS_PER_BATCH, D),
                                       jnp.float32),
    )(ts2, ts_blocks, in3)


@functools.partial(jax.jit, static_argnames=())
def kernel(input_encoded, timesteps, pos_encoding):
    in2d = input_encoded.reshape(N_ROWS, D)
    ts1d = timesteps.reshape(N_ROWS)

    out_sc = _run_sc(in2d, ts1d, pos_encoding)

    in3 = input_encoded.reshape(B, ROWS_PER_BATCH, D)
    ts2 = timesteps.reshape(B, 1, ROWS_PER_BATCH)
    ts_blocks = timesteps.reshape(B * (ROWS_PER_BATCH // TC_BLK), 1, TC_BLK)
    out_tc = _run_tc(in3, ts2, ts_blocks)

    out = jnp.concatenate(
        [out_sc.reshape(SC_BATCHES, ROWS_PER_BATCH, D), out_tc], axis=0)
    return out.reshape(B, S, T, D)


# speedup vs baseline: 1.4273x; 1.4273x over previous
import functools

import jax
import jax.numpy as jnp
from jax import lax
from jax.experimental import pallas as pl
from jax.experimental.pallas import tpu as pltpu
from jax.experimental.pallas import tpu_sc as plsc

B, S, T, D, L = 64, 50, 200, 64, 5000
ROWS_PER_BATCH = S * T
N_ROWS = B * ROWS_PER_BATCH
NUM_WORKERS = 32
BATCHES_PER_WORKER = B // NUM_WORKERS
CHUNK = 400
GCHUNK = 80
NGATHER = CHUNK // GCHUNK
NCHUNKS = ROWS_PER_BATCH // CHUNK
LANES = 16


def _sc_body(in_hbm, ts_hbm, table_hbm, out_hbm,
             ts_buf, idx_bufs, in_bufs, pe_buf, table_spm,
             sem_in, sem_out, sem_g):
    wid = lax.axis_index("s") * 2 + lax.axis_index("c")

    @pl.when(lax.axis_index("s") == 0)
    def _():
        pltpu.sync_copy(table_hbm, table_spm)

    plsc.subcore_barrier()

    for bb in range(BATCHES_PER_WORKER):
        b = wid * BATCHES_PER_WORKER + bb
        row0 = b * ROWS_PER_BATCH

        pltpu.sync_copy(ts_hbm.at[pl.ds(row0, ROWS_PER_BATCH)], ts_buf)

        def min_step(j, m):
            return jnp.minimum(m, ts_buf[pl.ds(j * LANES, LANES)])

        m0 = jnp.full((LANES,), jnp.iinfo(jnp.int32).max, dtype=jnp.int32)
        m = lax.fori_loop(0, ROWS_PER_BATCH // LANES, min_step, m0)
        iota = lax.broadcasted_iota(jnp.int32, (LANES,), 0)
        for k in (8, 4, 2, 1):
            perm = jnp.take_along_axis(m, iota ^ k, axis=0,
                                       mode="promise_in_bounds")
            m = jnp.minimum(m, perm)
        min_splat = m

        def compute_idx(j, p):
            for u in range(CHUNK // LANES):
                idx_bufs[p][pl.ds(u * LANES, LANES)] = (
                    ts_buf[pl.ds(j * CHUNK + u * LANES, LANES)] - min_splat)

        def in_copy(j, p):
            return pltpu.make_async_copy(
                in_hbm.at[pl.ds((row0 + j * CHUNK) * D, CHUNK * D)],
                in_bufs[p], sem_in[p])

        def out_copy(j, p):
            return pltpu.make_async_copy(
                in_bufs[p],
                out_hbm.at[pl.ds((row0 + j * CHUNK) * D, CHUNK * D)],
                sem_out[p])

        def chunk_body(j, p, first):
            in_copy(j, p).wait()
            descs = [
                pltpu.make_async_copy(
                    table_spm.at[idx_bufs[p].at[pl.ds(g * GCHUNK, GCHUNK)]],
                    pe_buf.at[pl.ds(g * GCHUNK, GCHUNK)],
                    sem_g)
                for g in range(NGATHER)
            ]
            for d in descs:
                d.start()
            q = 1 - p

            @pl.when(j + 1 < NCHUNKS)
            def _():
                if not first:
                    out_copy(j, q).wait()
                compute_idx(j + 1, q)
                in_copy(j + 1, q).start()

            for d in descs:
                d.wait()

            def add_rows(r, carry2):
                for rr in range(2):
                    for u in range(D // LANES):
                        fl = pl.ds((2 * r + rr) * D + u * LANES, LANES)
                        in_bufs[p][fl] = (
                            in_bufs[p][fl]
                            + pe_buf[2 * r + rr, pl.ds(u * LANES, LANES)])
                return carry2

            lax.fori_loop(0, CHUNK // 2, add_rows, 0)
            out_copy(j, p).start()

        compute_idx(0, 0)
        in_copy(0, 0).start()
        chunk_body(0, 0, first=True)

        def pair(i, carry):
            chunk_body(2 * i - 1, 1, first=False)
            chunk_body(2 * i, 0, first=False)
            return carry

        lax.fori_loop(1, (NCHUNKS + 1) // 2, pair, 0)

        out_copy(NCHUNKS - 2, 1).wait()
        out_copy(NCHUNKS - 1, 0).wait()


@functools.partial(jax.jit, static_argnames=())
def kernel(input_encoded, timesteps, pos_encoding):
    in1d = input_encoded.reshape(N_ROWS * D)
    ts1d = timesteps.reshape(N_ROWS)

    mesh = plsc.VectorSubcoreMesh(core_axis_name="c", subcore_axis_name="s")
    run = pl.kernel(
        _sc_body,
        out_type=jax.ShapeDtypeStruct((N_ROWS * D,), jnp.float32),
        mesh=mesh,
        scratch_types=[
            pltpu.VMEM((ROWS_PER_BATCH,), jnp.int32),
            [pltpu.VMEM((CHUNK,), jnp.int32) for _ in range(2)],
            [pltpu.VMEM((CHUNK * D,), jnp.float32) for _ in range(2)],
            pltpu.VMEM((CHUNK, D), jnp.float32),
            pltpu.VMEM_SHARED((L, D), jnp.float32),
            [pltpu.SemaphoreType.DMA for _ in range(2)],
            [pltpu.SemaphoreType.DMA for _ in range(2)],
            pltpu.SemaphoreType.DMA,
        ],
        compiler_params=pltpu.CompilerParams(use_tc_tiling_on_sc=False),
    )
    out1d = run(in1d, ts1d, pos_encoding)
    return out1d.reshape(B, S, T, D)


# speedup vs baseline: 1.4782x; 1.0356x over previous
import functools

import jax
import jax.numpy as jnp
from jax import lax
from jax.experimental import pallas as pl
from jax.experimental.pallas import tpu as pltpu
from jax.experimental.pallas import tpu_sc as plsc

B, S, T, D, L = 64, 50, 200, 64, 5000
ROWS_PER_BATCH = S * T
N_ROWS = B * ROWS_PER_BATCH
NUM_WORKERS = 32
BATCHES_PER_WORKER = B // NUM_WORKERS
CHUNK = 400
GCHUNK = 80
NGATHER = CHUNK // GCHUNK
NCHUNKS = ROWS_PER_BATCH // CHUNK
LANES = 16


def _sc_body(in_hbm, ts_hbm, table_hbm, out_hbm,
             ts_buf, idx_bufs, in_bufs, table_spm,
             sem_in, sem_out, sem_g):
    wid = lax.axis_index("s") * 2 + lax.axis_index("c")

    @pl.when(lax.axis_index("s") == 0)
    def _():
        pltpu.sync_copy(table_hbm, table_spm)

    plsc.subcore_barrier()

    for bb in range(BATCHES_PER_WORKER):
        b = wid * BATCHES_PER_WORKER + bb
        row0 = b * ROWS_PER_BATCH

        pltpu.sync_copy(ts_hbm.at[pl.ds(row0, ROWS_PER_BATCH)], ts_buf)

        def min_step(j, m):
            return jnp.minimum(m, ts_buf[pl.ds(j * LANES, LANES)])

        m0 = jnp.full((LANES,), jnp.iinfo(jnp.int32).max, dtype=jnp.int32)
        m = lax.fori_loop(0, ROWS_PER_BATCH // LANES, min_step, m0)
        iota = lax.broadcasted_iota(jnp.int32, (LANES,), 0)
        for k in (8, 4, 2, 1):
            perm = jnp.take_along_axis(m, iota ^ k, axis=0,
                                       mode="promise_in_bounds")
            m = jnp.minimum(m, perm)
        min_splat = m

        def compute_idx(j, p):
            for u in range(CHUNK // LANES):
                idx_bufs[p][pl.ds(u * LANES, LANES)] = (
                    ts_buf[pl.ds(j * CHUNK + u * LANES, LANES)] - min_splat)

        def in_copy(j, p):
            return pltpu.make_async_copy(
                in_hbm.at[pl.ds(row0 + j * CHUNK, CHUNK)],
                in_bufs[p], sem_in[p])

        def out_copy(j, p):
            return pltpu.make_async_copy(
                in_bufs[p], out_hbm.at[pl.ds(row0 + j * CHUNK, CHUNK)],
                sem_out[p])

        def chunk_body(j, p, first):
            in_copy(j, p).wait()
            descs = [
                pltpu.make_async_copy(
                    table_spm.at[idx_bufs[p].at[pl.ds(g * GCHUNK, GCHUNK)]],
                    in_bufs[p].at[pl.ds(g * GCHUNK, GCHUNK)],
                    sem_g)
                for g in range(NGATHER)
            ]
            for d in descs:
                d.start(add=True)
            q = 1 - p

            @pl.when(j + 1 < NCHUNKS)
            def _():
                if not first:
                    out_copy(j, q).wait()
                compute_idx(j + 1, q)
                in_copy(j + 1, q).start()

            for d in descs:
                d.wait()

            out_copy(j, p).start()

        compute_idx(0, 0)
        in_copy(0, 0).start()
        chunk_body(0, 0, first=True)

        def pair(i, carry):
            chunk_body(2 * i - 1, 1, first=False)
            chunk_body(2 * i, 0, first=False)
            return carry

        lax.fori_loop(1, (NCHUNKS + 1) // 2, pair, 0)

        out_copy(NCHUNKS - 2, 1).wait()
        out_copy(NCHUNKS - 1, 0).wait()


@functools.partial(jax.jit, static_argnames=())
def kernel(input_encoded, timesteps, pos_encoding):
    in2d = input_encoded.reshape(N_ROWS, D)
    ts1d = timesteps.reshape(N_ROWS)

    mesh = plsc.VectorSubcoreMesh(core_axis_name="c", subcore_axis_name="s")
    run = pl.kernel(
        _sc_body,
        out_type=jax.ShapeDtypeStruct((N_ROWS, D), jnp.float32),
        mesh=mesh,
        scratch_types=[
            pltpu.VMEM((ROWS_PER_BATCH,), jnp.int32),
            [pltpu.VMEM((CHUNK,), jnp.int32) for _ in range(2)],
            [pltpu.VMEM((CHUNK, D), jnp.float32) for _ in range(2)],
            pltpu.VMEM_SHARED((L, D), jnp.float32),
            [pltpu.SemaphoreType.DMA for _ in range(2)],
            [pltpu.SemaphoreType.DMA for _ in range(2)],
            pltpu.SemaphoreType.DMA,
        ],
        compiler_params=pltpu.CompilerParams(use_tc_tiling_on_sc=False),
    )
    out2d = run(in2d, ts1d, pos_encoding)
    return out2d.reshape(B, S, T, D)
